# Initial kernel scaffold; baseline (speedup 1.0000x reference)
import jax
import jax.numpy as jnp
from jax.experimental import pallas as pl


def kernel(x, edge_index, W_rel1, b_rel1, W_root1, p1, W_rel2, b_rel2, W_root2, p2, W_rel3, b_rel3, W_root3, p3, W_lin1, b_lin1, W_lin2, b_lin2, W_lin3, b_lin3):
    raise NotImplementedError("write your pallas kernel here")


# speedup vs baseline: 71.4743x; 71.4743x over previous
import functools
import math

import jax
import jax.numpy as jnp
from jax import lax
from jax.experimental import pallas as pl
from jax.experimental.pallas import tpu as pltpu
from jax.experimental.pallas import tpu_sc as plsc

F = 128
BLK = 512
NC, NS, L = 2, 16, 16
NW = NC * NS


def _ceil_to(x, m):
    return ((x + m - 1) // m) * m


def _d1_body(n, agg_ref, x_ref, wr_ref, wo_ref, b_ref, p_ref,
             h_ref, s_ref):
    i = pl.program_id(0)
    a = agg_ref[0] + agg_ref[1]
    h = jnp.dot(a, wr_ref[...], preferred_element_type=jnp.float32)
    h += jnp.dot(x_ref[...], wo_ref[...], preferred_element_type=jnp.float32)
    h = jnp.maximum(h + b_ref[...][None, :], 0.0)
    h_ref[...] = h
    p = p_ref[...]
    pn = p / jnp.sqrt(jnp.sum(p * p))
    s = lax.dot_general(h, pn.reshape(128, 1), (((1,), (0,)), ((), ())),
                        preferred_element_type=jnp.float32)
    s = jnp.tanh(s)
    rows = i * BLK + lax.broadcasted_iota(jnp.int32, (BLK, 1), 0)
    s_ref[...] = jnp.where(rows < n, s, -2.0)


def _dense1(aggp, xp, wr, wo, b, p, n, npad):
    grid = npad // BLK
    return pl.pallas_call(
        functools.partial(_d1_body, n),
        grid=(grid,),
        in_specs=[
            pl.BlockSpec((2, BLK, 4), lambda i: (0, i, 0)),
            pl.BlockSpec((BLK, 4), lambda i: (i, 0)),
            pl.BlockSpec((4, 128), lambda i: (0, 0)),
            pl.BlockSpec((4, 128), lambda i: (0, 0)),
            pl.BlockSpec((128,), lambda i: (0,)),
            pl.BlockSpec((128,), lambda i: (0,)),
        ],
        out_specs=[
            pl.BlockSpec((BLK, 128), lambda i: (i, 0)),
            pl.BlockSpec((BLK, 1), lambda i: (i, 0)),
        ],
        out_shape=[
            jax.ShapeDtypeStruct((npad, 128), jnp.float32),
            jax.ShapeDtypeStruct((npad, 1), jnp.float32),
        ],
    )(aggp, xp, wr, wo, b, p)


def _d2_body(n, kprev, nblk, agg_ref, xn_ref, wr_ref, wo_ref, b_ref, p_ref,
             h_ref, s_ref, xo_ref, mx_ref, sm_ref):
    i = pl.program_id(0)
    a = agg_ref[...]
    xn = xn_ref[...]
    h = jnp.dot(a, wr_ref[...], preferred_element_type=jnp.float32)
    h += jnp.dot(xn, wo_ref[...], preferred_element_type=jnp.float32)
    h = jnp.maximum(h + b_ref[...][None, :], 0.0)
    h_ref[...] = h
    p = p_ref[...]
    pn = p / jnp.sqrt(jnp.sum(p * p))
    s = jnp.tanh(lax.dot_general(h, pn.reshape(128, 1),
                                 (((1,), (0,)), ((), ())),
                                 preferred_element_type=jnp.float32))
    rows1 = i * BLK + lax.broadcasted_iota(jnp.int32, (BLK, 1), 0)
    s_ref[...] = jnp.where(rows1 < n, s, -2.0)
    rows = i * BLK + lax.broadcasted_iota(jnp.int32, (BLK, 128), 0)
    valid = rows < kprev
    xm = jnp.where(valid, xn, -jnp.inf)
    xs = jnp.where(valid, xn, 0.0)
    bmax = jnp.max(xm, axis=0, keepdims=True)
    bsum = jnp.sum(xs, axis=0, keepdims=True)

    @pl.when(i == 0)
    def _():
        mx_ref[...] = bmax
        sm_ref[...] = bsum

    @pl.when(i > 0)
    def _():
        mx_ref[...] = jnp.maximum(mx_ref[...], bmax)
        sm_ref[...] = sm_ref[...] + bsum

    @pl.when(i == nblk - 1)
    def _():
        xo_ref[...] = jnp.concatenate(
            [mx_ref[...], sm_ref[...] / float(kprev)], axis=1)


def _dense2(aggp, xn, wr, wo, b, p, n, kprev, npad):
    nblk = npad // BLK
    return pl.pallas_call(
        functools.partial(_d2_body, n, kprev, nblk),
        grid=(nblk,),
        in_specs=[
            pl.BlockSpec((BLK, 128), lambda i: (i, 0)),
            pl.BlockSpec((BLK, 128), lambda i: (i, 0)),
            pl.BlockSpec((128, 128), lambda i: (0, 0)),
            pl.BlockSpec((128, 128), lambda i: (0, 0)),
            pl.BlockSpec((128,), lambda i: (0,)),
            pl.BlockSpec((128,), lambda i: (0,)),
        ],
        out_specs=[
            pl.BlockSpec((BLK, 128), lambda i: (i, 0)),
            pl.BlockSpec((BLK, 1), lambda i: (i, 0)),
            pl.BlockSpec((1, 256), lambda i: (0, 0)),
        ],
        out_shape=[
            jax.ShapeDtypeStruct((npad, 128), jnp.float32),
            jax.ShapeDtypeStruct((npad, 1), jnp.float32),
            jax.ShapeDtypeStruct((1, 256), jnp.float32),
        ],
        scratch_shapes=[
            pltpu.VMEM((1, 128), jnp.float32),
            pltpu.VMEM((1, 128), jnp.float32),
        ],
    )(aggp, xn, wr, wo, b, p)


def _f_body(k3, xn_ref, x1_ref, x2_ref, w1_ref, b1_ref, w2_ref, b2_ref,
            w3_ref, b3_ref, o_ref):
    xn = xn_ref[...]
    rows = lax.broadcasted_iota(jnp.int32, xn.shape, 0)
    valid = rows < k3
    mx = jnp.max(jnp.where(valid, xn, -jnp.inf), axis=0, keepdims=True)
    sm = jnp.sum(jnp.where(valid, xn, 0.0), axis=0, keepdims=True)
    x3 = jnp.concatenate([mx, sm / float(k3)], axis=1)
    z = x1_ref[...] + x2_ref[...] + x3
    z = jnp.maximum(jnp.dot(z, w1_ref[...],
                            preferred_element_type=jnp.float32)
                    + b1_ref[...][None, :], 0.0)
    z = jnp.maximum(jnp.dot(z, w2_ref[...],
                            preferred_element_type=jnp.float32)
                    + b2_ref[...][None, :], 0.0)
    z = jnp.dot(z, w3_ref[...], preferred_element_type=jnp.float32) \
        + b3_ref[...][None, :]
    o_ref[...] = jax.nn.sigmoid(z)


def _final(xn3, x1, x2, w1, b1, w2, b2, w3, b3, k3):
    return pl.pallas_call(
        functools.partial(_f_body, k3),
        out_shape=jax.ShapeDtypeStruct((1, 9), jnp.float32),
    )(xn3, x1, x2, w1, b1, w2, b2, w3, b3)


def _scatter1_sc(xf, src, dst, zf, npad):
    E = src.shape[0]
    EW = E // NW
    B = 2000
    CH4 = (npad // NS) * 4
    mesh = plsc.VectorSubcoreMesh(core_axis_name="c", subcore_axis_name="s",
                                  num_cores=NC, num_subcores=NS)
    NF = npad * 4

    @functools.partial(
        pl.kernel,
        out_type=jax.ShapeDtypeStruct((2, NF), jnp.float32),
        mesh=mesh,
        compiler_params=pltpu.CompilerParams(needs_layout_passes=False),
        scratch_types=[
            pltpu.VMEM_SHARED((NF,), jnp.float32),
            pltpu.VMEM_SHARED((NF,), jnp.float32),
            pltpu.VMEM((B,), jnp.int32),
            pltpu.VMEM((B,), jnp.int32),
            [pltpu.VMEM((B,), jnp.int32) for _ in range(4)],
            [pltpu.VMEM((B,), jnp.int32) for _ in range(4)],
            [pltpu.VMEM((B,), jnp.float32) for _ in range(4)],
            pltpu.SemaphoreType.DMA,
            pltpu.SemaphoreType.DMA,
        ],
    )
    def body(se_hbm, de_hbm, x_hbm, z_hbm, out_hbm, x_s, agg_s, src_v,
             dst_v, isb, idb, val, sem1, sem2):
        c = lax.axis_index("c")
        s = lax.axis_index("s")
        wid = c * NS + s
        pltpu.sync_copy(x_hbm.at[pl.ds(s * CH4, CH4)],
                        x_s.at[pl.ds(s * CH4, CH4)])
        pltpu.sync_copy(z_hbm.at[pl.ds(s * CH4, CH4)],
                        agg_s.at[pl.ds(s * CH4, CH4)])
        plsc.subcore_barrier()

        def chunk(i, carry):
            b = wid * EW + i * B
            pltpu.sync_copy(se_hbm.at[pl.ds(b, B)], src_v)
            pltpu.sync_copy(de_hbm.at[pl.ds(b, B)], dst_v)

            def vl(v, carry2):
                sv = src_v[pl.ds(v * 16, 16)] * 4
                dv = dst_v[pl.ds(v * 16, 16)] * 4
                for j in range(4):
                    isb[j][pl.ds(v * 16, 16)] = sv + j
                    idb[j][pl.ds(v * 16, 16)] = dv + j
                return carry2
            lax.fori_loop(0, B // 16, vl, 0)
            gcopies = [pltpu.async_copy(x_s.at[isb[j]], val[j], sem1)
                       for j in range(4)]
            for g in gcopies:
                g.wait()
            scopies = [pltpu.async_copy(val[j], agg_s.at[idb[j]], sem2,
                                        add=True) for j in range(4)]
            for g in scopies:
                g.wait()
            return carry

        lax.fori_loop(0, EW // B, chunk, 0)
        plsc.subcore_barrier()
        pltpu.sync_copy(agg_s.at[pl.ds(s * CH4, CH4)],
                        out_hbm.at[c, pl.ds(s * CH4, CH4)])

    return body(src, dst, xf, zf)


def _topk_sc(score, h, zrows, n, k, npad, kpad):
    SW = npad // NS
    NV = SW // 16
    G2 = 128
    k_dn = (k // 8) * 8
    PZ = kpad - k_dn
    mesh = plsc.VectorSubcoreMesh(core_axis_name="c", subcore_axis_name="s",
                                  num_cores=NC, num_subcores=NS)

    @functools.partial(
        pl.kernel,
        out_type=[jax.ShapeDtypeStruct((npad,), jnp.int32),
                  jax.ShapeDtypeStruct((kpad, 128), jnp.float32)],
        mesh=mesh,
        compiler_params=pltpu.CompilerParams(needs_layout_passes=False),
        scratch_types=[
            pltpu.VMEM((SW,), jnp.float32),
            pltpu.VMEM((SW,), jnp.uint32),
            pltpu.VMEM((4096,), jnp.int32),
            pltpu.VMEM((256,), jnp.int32),
            pltpu.VMEM_SHARED((16, 256), jnp.int32),
            pltpu.VMEM_SHARED((16, 16), jnp.int32),
            pltpu.VMEM((16, 256), jnp.int32),
            pltpu.VMEM((16, 16), jnp.int32),
            pltpu.VMEM((16,), jnp.int32),
            pltpu.VMEM((SW + 16,), jnp.int32),
            pltpu.VMEM((SW + 16,), jnp.int32),
            pltpu.VMEM((SW + 16,), jnp.float32),
            pltpu.VMEM((SW,), jnp.int32),
            pltpu.VMEM((G2, 128), jnp.float32),
            pltpu.SemaphoreType.DMA,
            pltpu.SemaphoreType.DMA,
        ],
    )
    def body(score_hbm, h_hbm, z_hbm, map_hbm, xn_hbm, sc_v, key_v, lh, wh,
             hist_s, cnt_s, gh2, cl_v, cw_v, selN, selI, selS, map_v,
             rows_v, sem1, sem2):
        i16 = lax.iota(jnp.int32, 16)
        u1 = jnp.uint32(0xFFFFFFFF)
        c = lax.axis_index("c")
        s = lax.axis_index("s")

        @pl.when(c == 0)
        def _core0():
            @pl.when(s == 0)
            def _():
                pltpu.sync_copy(z_hbm.at[pl.ds(0, PZ), :],
                                xn_hbm.at[pl.ds(k_dn, PZ), :])

            wbase = s * SW
            pltpu.sync_copy(score_hbm.at[pl.ds(wbase, SW)], sc_v)

            def keyl(g, carry):
                f = sc_v[pl.ds(g * 16, 16)]
                u = plsc.bitcast(f, jnp.uint32)
                neg = (u >> 31) == 1
                key_v[pl.ds(g * 16, 16)] = jnp.where(
                    neg, u ^ u1, u | jnp.uint32(0x80000000))
                return carry
            lax.fori_loop(0, NV, keyl, 0)

            pref = jnp.uint32(0)
            kk = jnp.int32(k)
            for r in range(4):
                sh = 24 - 8 * r
                if r == 0:
                    mskc = jnp.uint32(0)
                else:
                    mskc = jnp.uint32((0xFFFFFFFF << (sh + 8)) & 0xFFFFFFFF)

                def zl(t, carry):
                    lh[pl.ds(t * 16, 16)] = jnp.zeros((16,), jnp.int32)
                    return carry
                lax.fori_loop(0, 256, zl, 0)

                def hl(g, carry, sh=sh, mskc=mskc, pref=pref):
                    mk = key_v[pl.ds(g * 16, 16)]
                    act = (mk & mskc) == (pref & mskc)
                    d = lax.convert_element_type(
                        (mk >> sh) & jnp.uint32(255), jnp.int32)
                    plsc.addupdate_scatter(
                        lh, [i16 * 256 + d], jnp.ones((16,), jnp.int32),
                        mask=act)
                    return carry
                lax.fori_loop(0, NV, hl, 0)

                def ml(dc, carry):
                    acc = jnp.zeros((16,), jnp.int32)
                    for l in range(16):
                        acc = acc + lh[pl.ds(l * 256 + dc * 16, 16)]
                    wh[pl.ds(dc * 16, 16)] = acc
                    return carry
                lax.fori_loop(0, 16, ml, 0)
                pltpu.sync_copy(wh, hist_s.at[s])
                plsc.subcore_barrier()
                pltpu.sync_copy(hist_s, gh2)
                tot_above = jnp.int32(0)
                found = jnp.bool_(False)
                d_star = jnp.int32(0)
                kk_new = kk
                for dc in range(15, -1, -1):
                    hv = jnp.zeros((16,), jnp.int32)
                    for w in range(16):
                        hv = hv + gh2[w, pl.ds(dc * 16, 16)]
                    hrev = lax.rev(hv, (0,))
                    rs = plsc.cumsum(hrev)
                    cum = tot_above + rs
                    m = cum >= kk
                    fl = jnp.min(jnp.where(m, i16, 16))
                    this = jnp.logical_and(fl < 16, jnp.logical_not(found))
                    cand = dc * 16 + 15 - fl
                    above = jnp.sum(jnp.where(i16 == fl, cum - hrev, 0))
                    d_star = jnp.where(this, cand, d_star)
                    kk_new = jnp.where(this, kk - above, kk_new)
                    found = jnp.logical_or(found, fl < 16)
                    tot_above = tot_above + jnp.sum(hv)
                pref = pref | (lax.convert_element_type(d_star, jnp.uint32)
                               << sh)
                kk = kk_new
                plsc.subcore_barrier()
            T = pref
            eq_quota = kk

            def cl(g, carry):
                lgt, leq = carry
                mk = key_v[pl.ds(g * 16, 16)]
                lgt = lgt + jnp.sum((mk > T).astype(jnp.int32))
                leq = leq + jnp.sum((mk == T).astype(jnp.int32))
                return (lgt, leq)
            lgt, leq = lax.fori_loop(0, NV, cl, (jnp.int32(0), jnp.int32(0)))
            cw_v[...] = (jnp.where(i16 == 0, lgt, 0)
                         + jnp.where(i16 == 1, leq, 0))
            pltpu.sync_copy(cw_v, cnt_s.at[s])
            plsc.subcore_barrier()
            pltpu.sync_copy(cnt_s, cl_v)
            z16 = jnp.zeros((16,), jnp.int32)
            gts = plsc.load_gather(cl_v, [i16, z16])
            eqs = plsc.load_gather(cl_v, [i16, z16 + 1])
            gt_before = jnp.sum(jnp.where(i16 < s, gts, 0))
            eq_before = jnp.sum(jnp.where(i16 < s, eqs, 0))
            total_gt = jnp.sum(gts)

            def il(t, carry):
                selN[pl.ds(t * 16, 16)] = z16
                selI[pl.ds(t * 16, 16)] = k + i16
                selS[pl.ds(t * 16, 16)] = jnp.zeros((16,), jnp.float32)
                return carry
            lax.fori_loop(0, (SW + 16) // 16, il, 0)

            def al(g, carry):
                gcnt, ecnt, scnt = carry
                mk = key_v[pl.ds(g * 16, 16)]
                m_gt = mk > T
                m_eq = mk == T
                csg = plsc.cumsum(m_gt.astype(jnp.int32))
                cse = plsc.cumsum(m_eq.astype(jnp.int32))
                erank = ecnt + cse - 1
                take = jnp.logical_and(m_eq, erank < eq_quota)
                nid = jnp.where(m_gt, gcnt + csg - 1,
                                jnp.where(take, total_gt + erank, -1))
                map_v[pl.ds(g * 16, 16)] = nid
                sel = jnp.logical_or(m_gt, take)
                pos = scnt + plsc.cumsum(sel.astype(jnp.int32)) - 1
                node = wbase + g * 16 + i16
                plsc.store_scatter(selN, [pos], node, mask=sel)
                plsc.store_scatter(selI, [pos], nid, mask=sel)
                plsc.store_scatter(selS, [pos], sc_v[pl.ds(g * 16, 16)],
                                   mask=sel)
                return (gcnt + jnp.sum(m_gt.astype(jnp.int32)),
                        ecnt + jnp.sum(m_eq.astype(jnp.int32)),
                        scnt + jnp.sum(sel.astype(jnp.int32)))
            _, _, scnt = lax.fori_loop(
                0, NV, al, (gt_before, eq_before, jnp.int32(0)))
            pltpu.sync_copy(map_v, map_hbm.at[pl.ds(wbase, SW)])

            def bl(b, carry):
                pltpu.async_copy(h_hbm.at[selN.at[pl.ds(b * G2, G2)]],
                                 rows_v, sem1).wait()

                def rl(t, carry2):
                    sv16 = selS[pl.ds(b * G2 + t * 16, 16)]
                    for rr in range(16):
                        sval = sv16[rr]
                        r = t * 16 + rr
                        for j in range(8):
                            v = rows_v[r, pl.ds(j * 16, 16)]
                            rows_v[r, pl.ds(j * 16, 16)] = v * sval
                    return carry2
                lax.fori_loop(0, G2 // 16, rl, 0)
                pltpu.async_copy(rows_v,
                                 xn_hbm.at[selI.at[pl.ds(b * G2, G2)]],
                                 sem2).wait()
                return carry
            lax.fori_loop(0, (scnt + G2 - 1) // G2, bl, 0)

    mapping, xn = body(score, h, zrows)
    return mapping, xn


def _edge_compact_sc(e_src, e_dst, mapping, map_len, kpad_in, kpad_out,
                     capw, ew):
    B = 2000
    NCH = ew // B
    mesh = plsc.VectorSubcoreMesh(core_axis_name="c", subcore_axis_name="s",
                                  num_cores=NC, num_subcores=NS)
    capsel = _ceil_to(capw, 16)

    @functools.partial(
        pl.kernel,
        out_type=[jax.ShapeDtypeStruct((NW * capw,), jnp.int32),
                  jax.ShapeDtypeStruct((NW * capw,), jnp.int32)],
        mesh=mesh,
        compiler_params=pltpu.CompilerParams(needs_layout_passes=False),
        scratch_types=[
            pltpu.VMEM((map_len,), jnp.int32),
            pltpu.VMEM((B,), jnp.int32),
            pltpu.VMEM((B,), jnp.int32),
            pltpu.VMEM((capsel,), jnp.int32),
            pltpu.VMEM((capsel,), jnp.int32),
        ],
    )
    def body(es_hbm, ed_hbm, map_hbm, cs_hbm, cd_hbm,
             map_t, se_v, de_v, selA, selB):
        i16 = lax.iota(jnp.int32, 16)
        c = lax.axis_index("c")
        s = lax.axis_index("s")
        wid = c * NS + s
        pltpu.sync_copy(map_hbm, map_t)

        def il(t, carry):
            selA[pl.ds(t * 16, 16)] = kpad_in - 16 + i16
            selB[pl.ds(t * 16, 16)] = jnp.full((16,), kpad_out, jnp.int32)
            return carry
        lax.fori_loop(0, capsel // 16, il, 0)

        def chunk(i, cnt):
            b = wid * ew + i * B
            pltpu.sync_copy(es_hbm.at[pl.ds(b, B)], se_v)
            pltpu.sync_copy(ed_hbm.at[pl.ds(b, B)], de_v)

            def vl(v, cnt2):
                sv = se_v[pl.ds(v * 16, 16)]
                dv = de_v[pl.ds(v * 16, 16)]
                ms = plsc.load_gather(map_t, [sv])
                md = plsc.load_gather(map_t, [dv])
                mok = jnp.logical_and(ms >= 0, md >= 0)
                pos = cnt2 + plsc.cumsum(mok.astype(jnp.int32)) - 1
                plsc.store_scatter(selA, [pos], ms, mask=mok)
                plsc.store_scatter(selB, [pos], md, mask=mok)
                return cnt2 + jnp.sum(mok.astype(jnp.int32))
            return lax.fori_loop(0, B // 16, vl, cnt)

        lax.fori_loop(0, NCH, chunk, jnp.int32(0))
        pltpu.sync_copy(selA.at[pl.ds(0, capw)],
                        cs_hbm.at[pl.ds(wid * capw, capw)])
        pltpu.sync_copy(selB.at[pl.ds(0, capw)],
                        cd_hbm.at[pl.ds(wid * capw, capw)])

    return body(e_src, e_dst, mapping)


def _gather_scatter_sc(cs, cd, mapping, map_len, xn, zagg, kpad_in,
                       kpad_out, ew):
    B = 2000
    G3 = 128
    NCH = ew // B
    HALF = kpad_out // 2
    DR = HALF // NS
    remap = mapping is not None
    mesh = plsc.VectorSubcoreMesh(core_axis_name="c", subcore_axis_name="s",
                                  num_cores=NC, num_subcores=NS)
    capsel = _ceil_to(ew, G3) + G3
    scratch = [
        pltpu.VMEM_SHARED((HALF, 128), jnp.float32),
        pltpu.VMEM((B,), jnp.int32),
        pltpu.VMEM((B,), jnp.int32),
        pltpu.VMEM((capsel,), jnp.int32),
        pltpu.VMEM((capsel,), jnp.int32),
        pltpu.VMEM((G3, 128), jnp.float32),
        pltpu.SemaphoreType.DMA,
        pltpu.SemaphoreType.DMA,
    ]
    if remap:
        scratch.insert(0, pltpu.VMEM((map_len + 16,), jnp.int32))

    @functools.partial(
        pl.kernel,
        out_type=jax.ShapeDtypeStruct((kpad_out, 128), jnp.float32),
        mesh=mesh,
        compiler_params=pltpu.CompilerParams(needs_layout_passes=False),
        scratch_types=scratch)
    def body(*refs):
        if remap:
            (cs_hbm, cd_hbm, map_hbm, xn_hbm, z_hbm, agg_hbm, map_t,
             agg_s, se_v, de_v, selA, selB, rows_v, sem1, sem2) = refs
        else:
            (cs_hbm, cd_hbm, xn_hbm, z_hbm, agg_hbm,
             agg_s, se_v, de_v, selA, selB, rows_v, sem1, sem2) = refs
        i16 = lax.iota(jnp.int32, 16)
        c = lax.axis_index("c")
        s = lax.axis_index("s")
        base_c = c * HALF
        if remap:
            pltpu.sync_copy(map_hbm, map_t.at[pl.ds(0, map_len)])
        pltpu.sync_copy(z_hbm.at[pl.ds(s * DR, DR), :],
                        agg_s.at[pl.ds(s * DR, DR), :])
        plsc.subcore_barrier()

        def il(t, carry):
            selA[pl.ds(t * 16, 16)] = kpad_in - 16 + i16
            selB[pl.ds(t * 16, 16)] = jnp.remainder(t * 16 + i16,
                                                    jnp.int32(HALF))
            return carry
        lax.fori_loop(0, capsel // 16, il, 0)

        def chunk(i, cnt):
            b = s * ew + i * B
            pltpu.sync_copy(cs_hbm.at[pl.ds(b, B)], se_v)
            pltpu.sync_copy(cd_hbm.at[pl.ds(b, B)], de_v)

            def vl(v, cnt2):
                sv = se_v[pl.ds(v * 16, 16)]
                dv = de_v[pl.ds(v * 16, 16)]
                if remap:
                    ms = plsc.load_gather(map_t, [sv])
                    md = plsc.load_gather(
                        map_t, [jnp.minimum(dv, map_len + 15)])
                    mdl = md - base_c
                    mok = jnp.logical_and(
                        jnp.logical_and(ms >= 0, mdl >= 0), mdl < HALF)
                else:
                    ms = sv
                    mdl = dv - base_c
                    mok = jnp.logical_and(mdl >= 0, mdl < HALF)
                pos = cnt2 + plsc.cumsum(mok.astype(jnp.int32)) - 1
                plsc.store_scatter(selA, [pos], ms, mask=mok)
                plsc.store_scatter(selB, [pos], mdl, mask=mok)
                return cnt2 + jnp.sum(mok.astype(jnp.int32))
            return lax.fori_loop(0, B // 16, vl, cnt)

        cnt = lax.fori_loop(0, NCH, chunk, jnp.int32(0))

        def bl(b, carry):
            pltpu.async_copy(xn_hbm.at[selA.at[pl.ds(b * G3, G3)]],
                             rows_v, sem1).wait()
            pltpu.async_copy(rows_v, agg_s.at[selB.at[pl.ds(b * G3, G3)]],
                             sem2, add=True).wait()
            return carry
        lax.fori_loop(0, (cnt + G3 - 1) // G3, bl, 0)
        plsc.subcore_barrier()
        pltpu.sync_copy(agg_s.at[pl.ds(s * DR, DR), :],
                        agg_hbm.at[pl.ds(base_c + s * DR, DR), :])

    if remap:
        return body(cs, cd, mapping, xn, zagg)
    return body(cs, cd, xn, zagg)


def kernel(x, edge_index, W_rel1, b_rel1, W_root1, p1, W_rel2, b_rel2,
           W_root2, p2, W_rel3, b_rel3, W_root3, p3, W_lin1, b_lin1,
           W_lin2, b_lin2, W_lin3, b_lin3):
    n1 = x.shape[0]
    k1 = int(math.ceil(0.25 * n1))
    k2 = int(math.ceil(0.25 * k1))
    k3 = int(math.ceil(0.25 * k2))
    np1 = _ceil_to(n1, BLK)
    kp1 = _ceil_to(k1, BLK)
    kp2 = _ceil_to(k2, BLK)
    kp3 = _ceil_to(k3, BLK)

    xp = jnp.pad(x, ((0, np1 - n1), (0, 0)))
    zr = jnp.zeros((512, 128), jnp.float32)
    E = edge_index.shape[1]
    CAPW = 8000

    agg1f = _scatter1_sc(xp.reshape(-1), edge_index[0], edge_index[1],
                         jnp.zeros((np1 * 4,), jnp.float32), np1)
    agg1p = agg1f.reshape(2, np1, 4)
    h1, s1 = _dense1(agg1p, xp, W_rel1, W_root1, b_rel1, p1, n1, np1)
    map1, xn1 = _topk_sc(s1.reshape(-1), h1, zr, n1, k1, np1, kp1)

    cs1, cd1 = _edge_compact_sc(edge_index[0], edge_index[1], map1,
                                np1, kp1, kp1, CAPW, E // NW)
    agg2 = _gather_scatter_sc(cs1, cd1, None, 0, xn1,
                              jnp.zeros((kp1 // 2, 128), jnp.float32),
                              kp1, kp1, 2 * CAPW)
    h2, s2, x1 = _dense2(agg2, xn1, W_rel2, W_root2, b_rel2, p2,
                         k1, k1, kp1)
    map2, xn2 = _topk_sc(s2.reshape(-1), h2, zr, k1, k2, kp1, kp2)

    agg3 = _gather_scatter_sc(cs1, cd1, map2, kp1, xn2,
                              jnp.zeros((kp2 // 2, 128), jnp.float32),
                              kp2, kp2, 2 * CAPW)
    h3, s3, x2 = _dense2(agg3, xn2, W_rel3, W_root3, b_rel3, p3,
                         k2, k2, kp2)
    _, xn3 = _topk_sc(s3.reshape(-1), h3, zr, k2, k3, kp2, kp3)

    return _final(xn3, x1, x2, W_lin1, b_lin1, W_lin2, b_lin2,
                  W_lin3, b_lin3, k3)
